# baseline (speedup 1.0000x reference)
import functools

import jax
import jax.numpy as jnp
from jax import lax
from jax.experimental import pallas as pl
from jax.experimental.pallas import tpu as pltpu
from jax.experimental.pallas import tpu_sc as plsc

N = 10000
D = 128
E = 320000

NW = 32
K = 256
NCH = 40
EPW = NCH * K
E_PAD = NW * EPW
N_PAD = 10512
RPT = N_PAD // 16
ZR = 73

_mesh = plsc.VectorSubcoreMesh(core_axis_name="c", subcore_axis_name="s")


def _sc_scatter(F):
  @functools.partial(
      pl.kernel,
      out_type=jax.ShapeDtypeStruct((2, N_PAD, F), jnp.float32),
      mesh=_mesh,
      compiler_params=pltpu.CompilerParams(use_tc_tiling_on_sc=False),
      scratch_types=[
          pltpu.VMEM((NCH + 2, K), jnp.int32),
          pltpu.VMEM((NCH + 1, K), jnp.int32),
          pltpu.VMEM((3, K, F), jnp.float32),
          pltpu.VMEM((ZR, F), jnp.float32),
          pltpu.VMEM_SHARED((N_PAD, F), jnp.float32),
          pltpu.SemaphoreType.DMA,
          pltpu.SemaphoreType.DMA,
          pltpu.SemaphoreType.DMA,
          pltpu.SemaphoreType.DMA,
          pltpu.SemaphoreType.DMA,
          pltpu.SemaphoreType.DMA,
      ],
  )
  def k(u_hbm, srcp_hbm, dstp_hbm, out_hbm, src_t, dst_t, rows, zbuf, acc,
        g0, g1, g2, s0, s1, s2):
    gsems = (g0, g1, g2)
    ssems = (s0, s1, s2)
    c = lax.axis_index("c")
    s = lax.axis_index("s")
    wid = s * 2 + c

    def zrow(r, carry):
      for t in range(F // 16):
        zbuf[r, pl.ds(t * 16, 16)] = jnp.zeros((16,), jnp.float32)
      return carry

    lax.fori_loop(0, ZR, zrow, 0)

    def zcp(i, carry):
      pltpu.sync_copy(zbuf, acc.at[pl.ds(s * RPT + i * ZR, ZR)])
      return carry

    lax.fori_loop(0, RPT // ZR, zcp, 0)

    pltpu.sync_copy(srcp_hbm.at[wid], src_t)
    pltpu.sync_copy(dstp_hbm.at[wid], dst_t)
    plsc.subcore_barrier()

    def gfire(j, b):
      pltpu.async_copy(u_hbm.at[src_t.at[j]], rows.at[b], gsems[b])

    def gwait(b):
      pltpu.make_async_copy(u_hbm.at[src_t.at[0]], rows.at[b],
                            gsems[b]).wait()

    def sfire(j, b):
      pltpu.async_copy(rows.at[b], acc.at[dst_t.at[j]], ssems[b], add=True)

    def swait(b):
      pltpu.make_async_copy(rows.at[b], acc.at[dst_t.at[0]], ssems[b]).wait()

    gfire(0, 0)
    gfire(1, 1)
    sfire(NCH, 2)

    def body(i, carry):
      j0 = 3 * i
      for t in range(3):
        j = j0 + t
        gwait(t)
        sfire(j, t)
        swait((t + 2) % 3)
        gfire(j + 2, (t + 2) % 3)
      return carry

    lax.fori_loop(0, (NCH - 1) // 3, body, 0)
    gwait(0)
    sfire(NCH - 1, 0)
    swait(2)
    gfire(NCH + 1, 2)
    swait(0)
    gwait(1)
    gwait(2)
    plsc.subcore_barrier()
    pltpu.sync_copy(acc.at[pl.ds(s * RPT, RPT)],
                    out_hbm.at[c, pl.ds(s * RPT, RPT)])

  return k


def _sc_degree():
  @functools.partial(
      pl.kernel,
      out_type=jax.ShapeDtypeStruct((2, N_PAD, 16), jnp.float32),
      mesh=_mesh,
      compiler_params=pltpu.CompilerParams(use_tc_tiling_on_sc=False),
      scratch_types=[
          pltpu.VMEM((NCH + 1, K), jnp.int32),
          pltpu.VMEM((K, 16), jnp.float32),
          pltpu.VMEM((ZR, 16), jnp.float32),
          pltpu.VMEM_SHARED((N_PAD, 16), jnp.float32),
          pltpu.SemaphoreType.DMA,
      ],
  )
  def k(dstp_hbm, out_hbm, dst_t, ones_b, zbuf, acc, ssem):
    c = lax.axis_index("c")
    s = lax.axis_index("s")
    wid = s * 2 + c

    def zrow(r, carry):
      zbuf[r, pl.ds(0, 16)] = jnp.zeros((16,), jnp.float32)
      return carry

    lax.fori_loop(0, ZR, zrow, 0)

    def orow(r, carry):
      ones_b[r, pl.ds(0, 16)] = jnp.ones((16,), jnp.float32)
      return carry

    lax.fori_loop(0, K, orow, 0)

    def zcp(i, carry):
      pltpu.sync_copy(zbuf, acc.at[pl.ds(s * RPT + i * ZR, ZR)])
      return carry

    lax.fori_loop(0, RPT // ZR, zcp, 0)

    pltpu.sync_copy(dstp_hbm.at[wid], dst_t)
    plsc.subcore_barrier()

    def body(i, carry):
      for b in range(8):
        pltpu.async_copy(ones_b, acc.at[dst_t.at[i * 8 + b]], ssem, add=True)
      for b in range(8):
        pltpu.make_async_copy(ones_b, acc.at[dst_t.at[0]], ssem).wait()
      return carry

    lax.fori_loop(0, NCH // 8, body, 0)
    plsc.subcore_barrier()
    pltpu.sync_copy(acc.at[pl.ds(s * RPT, RPT)],
                    out_hbm.at[c, pl.ds(s * RPT, RPT)])

  return k


_B = 10000


def _tc_mm_body(x_ref, W_ref, o_ref):
  o_ref[...] = jnp.dot(x_ref[...], W_ref[...],
                       preferred_element_type=jnp.float32)


def _tc_scale_body(p_ref, dpA, dpB, ou_ref, od_ref):
  deg = dpA[0][:, :1] + dpB[0][:, :1] + 1.0
  dinv = 1.0 / jnp.sqrt(deg)
  ou_ref[...] = dinv * p_ref[...]
  od_ref[...] = jnp.broadcast_to(dinv, od_ref.shape)


def _tc_mid_body(spA, spB, u_ref, dv_ref, W_ref, b_ref, o_ref):
  dinv = dv_ref[:, :1]
  h = jnp.maximum(dinv * (spA[0] + spB[0] + u_ref[...]) + b_ref[:1], 0.0)
  o_ref[...] = dinv * jnp.dot(h, W_ref[...], preferred_element_type=jnp.float32)


def _tc_out_body(spA, spB, u_ref, dv_ref, b_ref, Wl_ref, bl_ref, o_ref):
  dinv = dv_ref[:, :1]
  h = dinv * (spA[0] + spB[0] + u_ref[...]) + b_ref[:1]
  o_ref[...] = jnp.dot(h, Wl_ref[...],
                       preferred_element_type=jnp.float32) + bl_ref[:1]


def _row_spec(Fdim):
  return pl.BlockSpec((_B, Fdim), lambda i: (i, 0))


def _part_spec(Fdim):
  n = Fdim

  def a(i):
    return (0, i, 0)

  def b(i):
    return (1, i, 0)

  return (pl.BlockSpec((1, _B, n), a), pl.BlockSpec((1, _B, n), b))


def _full_spec(shape):
  nd = len(shape)
  return pl.BlockSpec(shape, lambda i: (0,) * nd)


def kernel(x, edge_index, W1, b1, W2, b2, W3, b3, Wl, bl):
  src = edge_index[0].astype(jnp.int32)
  dst = edge_index[1].astype(jnp.int32)

  pad = E_PAD - E
  ar = jnp.arange(pad, dtype=jnp.int32)
  dstp = jnp.concatenate([dst, N + (ar & 511)]).reshape(NW, NCH, K)
  dead = (N + (jnp.arange(NW * K, dtype=jnp.int32) & 511)).reshape(NW, 1, K)
  dst3 = jnp.concatenate([dstp, dead], axis=1)
  dst3 = lax.optimization_barrier(dst3)

  degp = _sc_degree()(dst3)

  srcp = jnp.concatenate([src, ar]).reshape(NW, NCH, K)
  ex = jnp.arange(NW * 2 * K, dtype=jnp.int32)
  extra = jnp.where(ex < N, ex, ex - N).reshape(NW, 2, K)
  src3 = jnp.concatenate([srcp, extra], axis=1)

  grid = (N // _B,)

  b1r = jnp.broadcast_to(b1[None, :], (8, b1.shape[0]))
  b2r = jnp.broadcast_to(b2[None, :], (8, b2.shape[0]))
  b3r = jnp.broadcast_to(b3[None, :], (8, b3.shape[0]))
  blr = jnp.broadcast_to(bl[None, :], (8, bl.shape[0]))

  p1 = pl.pallas_call(
      _tc_mm_body,
      grid=grid,
      in_specs=[_row_spec(D), _full_spec(W1.shape)],
      out_specs=_row_spec(64),
      out_shape=jax.ShapeDtypeStruct((N, 64), jnp.float32),
  )(x, W1)

  u1, dv = pl.pallas_call(
      _tc_scale_body,
      grid=grid,
      in_specs=[_row_spec(64), *_part_spec(16)],
      out_specs=[_row_spec(64), _row_spec(16)],
      out_shape=[jax.ShapeDtypeStruct((N, 64), jnp.float32),
                 jax.ShapeDtypeStruct((N, 16), jnp.float32)],
  )(p1, degp, degp)

  s1 = _sc_scatter(64)(u1, src3, dst3)

  u2 = pl.pallas_call(
      _tc_mid_body,
      grid=grid,
      in_specs=[*_part_spec(64), _row_spec(64), _row_spec(16),
                _full_spec(W2.shape), _full_spec((8, 64))],
      out_specs=_row_spec(32),
      out_shape=jax.ShapeDtypeStruct((N, 32), jnp.float32),
  )(s1, s1, u1, dv, W2, b1r)

  s2 = _sc_scatter(32)(u2, src3, dst3)

  u3 = pl.pallas_call(
      _tc_mid_body,
      grid=grid,
      in_specs=[*_part_spec(32), _row_spec(32), _row_spec(16),
                _full_spec(W3.shape), _full_spec((8, 32))],
      out_specs=_row_spec(16),
      out_shape=jax.ShapeDtypeStruct((N, 16), jnp.float32),
  )(s2, s2, u2, dv, W3, b2r)

  s3 = _sc_scatter(16)(u3, src3, dst3)

  out = pl.pallas_call(
      _tc_out_body,
      grid=grid,
      in_specs=[*_part_spec(16), _row_spec(16), _row_spec(16),
                _full_spec((8, 16)), _full_spec(Wl.shape), _full_spec((8, 7))],
      out_specs=_row_spec(7),
      out_shape=jax.ShapeDtypeStruct((N, 7), jnp.float32),
  )(s3, s3, u3, dv, b3r, Wl, blr)

  return out


# speedup vs baseline: 1.0158x; 1.0158x over previous
import functools

import jax
import jax.numpy as jnp
from jax import lax
from jax.experimental import pallas as pl
from jax.experimental.pallas import tpu as pltpu
from jax.experimental.pallas import tpu_sc as plsc

N = 10000
D = 128
E = 320000

NW = 32
K = 256
NCH = 40
EPW = NCH * K
E_PAD = NW * EPW
N_PAD = 10512
RPT = N_PAD // 16
ZR = 73

_mesh = plsc.VectorSubcoreMesh(core_axis_name="c", subcore_axis_name="s")


def _sc_scatter(F, Kc, NC):
  @functools.partial(
      pl.kernel,
      out_type=jax.ShapeDtypeStruct((2, N_PAD, F), jnp.float32),
      mesh=_mesh,
      compiler_params=pltpu.CompilerParams(use_tc_tiling_on_sc=False),
      scratch_types=[
          pltpu.VMEM((NC + 2, Kc), jnp.int32),
          pltpu.VMEM((NC + 1, Kc), jnp.int32),
          pltpu.VMEM((3, Kc, F), jnp.float32),
          pltpu.VMEM((ZR, F), jnp.float32),
          pltpu.VMEM_SHARED((N_PAD, F), jnp.float32),
          pltpu.SemaphoreType.DMA,
          pltpu.SemaphoreType.DMA,
          pltpu.SemaphoreType.DMA,
          pltpu.SemaphoreType.DMA,
          pltpu.SemaphoreType.DMA,
          pltpu.SemaphoreType.DMA,
      ],
  )
  def k(u_hbm, srcp_hbm, dstp_hbm, out_hbm, src_t, dst_t, rows, zbuf, acc,
        g0, g1, g2, s0, s1, s2):
    gsems = (g0, g1, g2)
    ssems = (s0, s1, s2)
    c = lax.axis_index("c")
    s = lax.axis_index("s")
    wid = s * 2 + c

    def zrow(r, carry):
      for t in range(F // 16):
        zbuf[r, pl.ds(t * 16, 16)] = jnp.zeros((16,), jnp.float32)
      return carry

    lax.fori_loop(0, ZR, zrow, 0)

    def zcp(i, carry):
      pltpu.sync_copy(zbuf, acc.at[pl.ds(s * RPT + i * ZR, ZR)])
      return carry

    lax.fori_loop(0, RPT // ZR, zcp, 0)

    pltpu.sync_copy(srcp_hbm.at[wid], src_t)
    pltpu.sync_copy(dstp_hbm.at[wid], dst_t)
    plsc.subcore_barrier()

    def gfire(j, b):
      pltpu.async_copy(u_hbm.at[src_t.at[j]], rows.at[b], gsems[b])

    def gwait(b):
      pltpu.make_async_copy(u_hbm.at[src_t.at[0]], rows.at[b],
                            gsems[b]).wait()

    def sfire(j, b):
      pltpu.async_copy(rows.at[b], acc.at[dst_t.at[j]], ssems[b], add=True)

    def swait(b):
      pltpu.make_async_copy(rows.at[b], acc.at[dst_t.at[0]], ssems[b]).wait()

    gfire(0, 0)
    gfire(1, 1)
    sfire(NC, 2)

    def body(i, carry):
      j0 = 3 * i
      for t in range(3):
        j = j0 + t
        gwait(t)
        sfire(j, t)
        swait((t + 2) % 3)
        gfire(j + 2, (t + 2) % 3)
      return carry

    lb = NC // 3
    lax.fori_loop(0, lb, body, 0)
    for j in range(3 * lb, NC):
      b = j % 3
      gwait(b)
      sfire(j, b)
      swait((b + 2) % 3)
      gfire(j + 2, (b + 2) % 3)
    swait((NC - 1) % 3)
    gwait(NC % 3)
    gwait((NC + 1) % 3)
    plsc.subcore_barrier()
    pltpu.sync_copy(acc.at[pl.ds(s * RPT, RPT)],
                    out_hbm.at[c, pl.ds(s * RPT, RPT)])

  return k


def _sc_degree():
  @functools.partial(
      pl.kernel,
      out_type=jax.ShapeDtypeStruct((2, N_PAD, 16), jnp.float32),
      mesh=_mesh,
      compiler_params=pltpu.CompilerParams(use_tc_tiling_on_sc=False),
      scratch_types=[
          pltpu.VMEM((NCH + 1, K), jnp.int32),
          pltpu.VMEM((K, 16), jnp.float32),
          pltpu.VMEM((ZR, 16), jnp.float32),
          pltpu.VMEM_SHARED((N_PAD, 16), jnp.float32),
          pltpu.SemaphoreType.DMA,
      ],
  )
  def k(dstp_hbm, out_hbm, dst_t, ones_b, zbuf, acc, ssem):
    c = lax.axis_index("c")
    s = lax.axis_index("s")
    wid = s * 2 + c

    def zrow(r, carry):
      zbuf[r, pl.ds(0, 16)] = jnp.zeros((16,), jnp.float32)
      return carry

    lax.fori_loop(0, ZR, zrow, 0)

    def orow(r, carry):
      ones_b[r, pl.ds(0, 16)] = jnp.ones((16,), jnp.float32)
      return carry

    lax.fori_loop(0, K, orow, 0)

    def zcp(i, carry):
      pltpu.sync_copy(zbuf, acc.at[pl.ds(s * RPT + i * ZR, ZR)])
      return carry

    lax.fori_loop(0, RPT // ZR, zcp, 0)

    pltpu.sync_copy(dstp_hbm.at[wid], dst_t)
    plsc.subcore_barrier()

    def body(i, carry):
      for b in range(8):
        pltpu.async_copy(ones_b, acc.at[dst_t.at[i * 8 + b]], ssem, add=True)
      for b in range(8):
        pltpu.make_async_copy(ones_b, acc.at[dst_t.at[0]], ssem).wait()
      return carry

    lax.fori_loop(0, NCH // 8, body, 0)
    plsc.subcore_barrier()
    pltpu.sync_copy(acc.at[pl.ds(s * RPT, RPT)],
                    out_hbm.at[c, pl.ds(s * RPT, RPT)])

  return k


_B = 10000


def _tc_mm_body(x_ref, W_ref, o_ref):
  o_ref[...] = jnp.dot(x_ref[...], W_ref[...],
                       preferred_element_type=jnp.float32)


def _tc_scale_body(p_ref, dpA, dpB, ou_ref, od_ref):
  deg = dpA[0][:, :1] + dpB[0][:, :1] + 1.0
  dinv = 1.0 / jnp.sqrt(deg)
  ou_ref[...] = dinv * p_ref[...]
  od_ref[...] = jnp.broadcast_to(dinv, od_ref.shape)


def _tc_mid_body(spA, spB, u_ref, dv_ref, W_ref, b_ref, o_ref):
  dinv = dv_ref[:, :1]
  h = jnp.maximum(dinv * (spA[0] + spB[0] + u_ref[...]) + b_ref[:1], 0.0)
  o_ref[...] = dinv * jnp.dot(h, W_ref[...], preferred_element_type=jnp.float32)


def _tc_out_body(spA, spB, u_ref, dv_ref, b_ref, Wl_ref, bl_ref, o_ref):
  dinv = dv_ref[:, :1]
  h = dinv * (spA[0] + spB[0] + u_ref[...]) + b_ref[:1]
  o_ref[...] = jnp.dot(h, Wl_ref[...],
                       preferred_element_type=jnp.float32) + bl_ref[:1]


def _row_spec(Fdim):
  return pl.BlockSpec((_B, Fdim), lambda i: (i, 0))


def _part_spec(Fdim):
  n = Fdim

  def a(i):
    return (0, i, 0)

  def b(i):
    return (1, i, 0)

  return (pl.BlockSpec((1, _B, n), a), pl.BlockSpec((1, _B, n), b))


def _full_spec(shape):
  nd = len(shape)
  return pl.BlockSpec(shape, lambda i: (0,) * nd)


def kernel(x, edge_index, W1, b1, W2, b2, W3, b3, Wl, bl):
  src = edge_index[0].astype(jnp.int32)
  dst = edge_index[1].astype(jnp.int32)

  pad = E_PAD - E
  ar = jnp.arange(pad, dtype=jnp.int32)
  dstp = jnp.concatenate([dst, N + (ar & 511)]).reshape(NW, NCH, K)
  dead = (N + (jnp.arange(NW * K, dtype=jnp.int32) & 511)).reshape(NW, 1, K)
  dst3 = jnp.concatenate([dstp, dead], axis=1)
  dst3 = lax.optimization_barrier(dst3)

  degp = _sc_degree()(dst3)

  srcp = jnp.concatenate([src, ar]).reshape(NW, NCH, K)
  ex = jnp.arange(NW * 2 * K, dtype=jnp.int32) & 8191
  extra = ex.reshape(NW, 2, K)
  src3 = jnp.concatenate([srcp, extra], axis=1)

  K5, NC5 = 512, NCH // 2
  src5 = jnp.concatenate(
      [jnp.concatenate([src, ar]).reshape(NW, NC5, K5),
       (jnp.arange(NW * 2 * K5, dtype=jnp.int32) & 8191).reshape(NW, 2, K5)],
      axis=1)
  dst5 = jnp.concatenate(
      [jnp.concatenate([dst, N + (ar & 511)]).reshape(NW, NC5, K5),
       (N + (jnp.arange(NW * K5, dtype=jnp.int32) & 511)).reshape(NW, 1, K5)],
      axis=1)

  grid = (N // _B,)

  b1r = jnp.broadcast_to(b1[None, :], (8, b1.shape[0]))
  b2r = jnp.broadcast_to(b2[None, :], (8, b2.shape[0]))
  b3r = jnp.broadcast_to(b3[None, :], (8, b3.shape[0]))
  blr = jnp.broadcast_to(bl[None, :], (8, bl.shape[0]))

  p1 = pl.pallas_call(
      _tc_mm_body,
      grid=grid,
      in_specs=[_row_spec(D), _full_spec(W1.shape)],
      out_specs=_row_spec(64),
      out_shape=jax.ShapeDtypeStruct((N, 64), jnp.float32),
  )(x, W1)

  u1, dv = pl.pallas_call(
      _tc_scale_body,
      grid=grid,
      in_specs=[_row_spec(64), *_part_spec(16)],
      out_specs=[_row_spec(64), _row_spec(16)],
      out_shape=[jax.ShapeDtypeStruct((N, 64), jnp.float32),
                 jax.ShapeDtypeStruct((N, 16), jnp.float32)],
  )(p1, degp, degp)

  s1 = _sc_scatter(64, K, NCH)(u1, src3, dst3)

  u2 = pl.pallas_call(
      _tc_mid_body,
      grid=grid,
      in_specs=[*_part_spec(64), _row_spec(64), _row_spec(16),
                _full_spec(W2.shape), _full_spec((8, 64))],
      out_specs=_row_spec(32),
      out_shape=jax.ShapeDtypeStruct((N, 32), jnp.float32),
  )(s1, s1, u1, dv, W2, b1r)

  s2 = _sc_scatter(32, K5, NC5)(u2, src5, dst5)

  u3 = pl.pallas_call(
      _tc_mid_body,
      grid=grid,
      in_specs=[*_part_spec(32), _row_spec(32), _row_spec(16),
                _full_spec(W3.shape), _full_spec((8, 32))],
      out_specs=_row_spec(16),
      out_shape=jax.ShapeDtypeStruct((N, 16), jnp.float32),
  )(s2, s2, u2, dv, W3, b2r)

  s3 = _sc_scatter(16, K5, NC5)(u3, src5, dst5)

  out = pl.pallas_call(
      _tc_out_body,
      grid=grid,
      in_specs=[*_part_spec(16), _row_spec(16), _row_spec(16),
                _full_spec((8, 16)), _full_spec(Wl.shape), _full_spec((8, 7))],
      out_specs=_row_spec(7),
      out_shape=jax.ShapeDtypeStruct((N, 7), jnp.float32),
  )(s3, s3, u3, dv, b3r, Wl, blr)

  return out
